# baseline (speedup 1.0000x reference)
import functools

import jax
import jax.numpy as jnp
from jax import lax
from jax.experimental import pallas as pl
from jax.experimental.pallas import tpu as pltpu
from jax.experimental.pallas import tpu_sc as plsc

BATCH = 16384
EMBED = 32
NROW = EMBED + 1
OROW = 40
V = 1000000
NUM_CORES = 2
NUM_SUBCORES = 16
LANES = 16
CW = 896
NFULL = 69
LASTW = 640
COLS_PER_TILE = NFULL * CW + LASTW
SLOTS = 8
SROW = 48
LIST_CAP = BATCH + LANES
TAIL_LO = 999936


def _scalar(vec, lane, iota):
    return jnp.sum(jnp.where(iota == lane, vec, jnp.zeros((LANES,), vec.dtype)))


@functools.partial(
    pl.kernel,
    mesh=plsc.VectorSubcoreMesh(core_axis_name="c", subcore_axis_name="s"),
    out_type=jax.ShapeDtypeStruct((NUM_CORES, OROW, BATCH), jnp.float32),
    compiler_params=pltpu.CompilerParams(needs_layout_passes=False),
    scratch_types=[
        pltpu.VMEM((2048,), jnp.int32),
        pltpu.VMEM((LIST_CAP,), jnp.int32),
        pltpu.VMEM((2, EMBED, CW + 1), jnp.float32),
        pltpu.VMEM((2, 1, CW), jnp.float32),
        pltpu.VMEM((1, SROW), jnp.float32),
        pltpu.VMEM((1, SROW), jnp.int32),
        pltpu.VMEM((LANES,), jnp.int32),
        pltpu.VMEM((V - TAIL_LO, EMBED), jnp.float32),
        pltpu.VMEM((V - TAIL_LO,), jnp.float32),
        pltpu.VMEM((2112,), jnp.float32),
        pltpu.VMEM_SHARED((NROW * BATCH,), jnp.float32),
        pltpu.SemaphoreType.DMA,
        pltpu.SemaphoreType.DMA,
        pltpu.SemaphoreType.DMA,
        pltpu.SemaphoreType.DMA,
    ],
)
def _mf_gather(user_hbm, item_hbm, uet_hbm, iet_hbm, ubt_hbm, ibt_hbm,
               ut_hbm, it_hbm, ubt2_hbm, ibt2_hbm,
               out_hbm, idx_v, lpk_v, cbuf, bbuf, srow_v, sidx_v,
               tmp_v, tail_v, tailb_v, zbuf, sh, csem0, csem1, ssem, msem):
    cid = lax.axis_index("c")
    tid = lax.axis_index("s")
    iota = lax.iota(jnp.int32, LANES)
    csems = (csem0, csem1)

    fz = jnp.zeros((LANES,), jnp.float32)

    def _zb(i, c):
        zbuf[pl.ds(i * LANES, LANES)] = fz
        return c

    lax.fori_loop(0, 2112 // LANES, _zb, 0)
    stripe_lo = tid * (NROW * BATCH // NUM_SUBCORES)

    def _zs(i, c):
        pltpu.sync_copy(zbuf, sh.at[pl.ds(stripe_lo + i * 2112, 2112)])
        return c

    lax.fori_loop(0, NROW * BATCH // NUM_SUBCORES // 2112, _zs, 0)
    plsc.subcore_barrier()

    def _scatter_entry(g0, g1, gb, pos, fired):
        srow_v[0, pl.ds(0, LANES)] = g0
        srow_v[0, pl.ds(LANES, LANES)] = g1
        srow_v[0, pl.ds(2 * LANES, LANES)] = gb * (1.0 / LANES)
        sidx_v[0, pl.ds(0, LANES)] = iota * BATCH + pos
        sidx_v[0, pl.ds(LANES, LANES)] = (iota + LANES) * BATCH + pos
        sidx_v[0, pl.ds(2 * LANES, LANES)] = \
            jnp.full((LANES,), EMBED * BATCH + pos, jnp.int32)
        pltpu.sync_copy(srow_v.at[0], sh.at[sidx_v.at[0]], add=True)
        return fired + 1

    def run_side(idx_hbm, tab_hbm, btab_hbm, ttab_hbm, tbias_hbm):
        tile_lo = tid * COLS_PER_TILE

        def fire(off, width, parity):
            offa = pl.multiple_of(off, 128)
            pltpu.async_copy(tab_hbm.at[:, pl.ds(offa, width)],
                             cbuf.at[parity, :, pl.ds(0, width)],
                             csems[parity])
            pltpu.async_copy(btab_hbm.at[:, pl.ds(offa, width)],
                             bbuf.at[parity, :, pl.ds(0, width)],
                             csems[parity])

        def drain(off, width, parity):
            offa = pl.multiple_of(off, 128)
            pltpu.make_async_copy(tab_hbm.at[:, pl.ds(offa, width)],
                                  cbuf.at[parity, :, pl.ds(0, width)],
                                  csems[parity]).wait()
            pltpu.make_async_copy(btab_hbm.at[:, pl.ds(offa, width)],
                                  bbuf.at[parity, :, pl.ds(0, width)],
                                  csems[parity]).wait()

        fire(tile_lo, CW, 0)
        fire(tile_lo + CW, CW, 1)

        tile_hi = jnp.where(tid == NUM_SUBCORES - 1, V,
                            tile_lo + COLS_PER_TILE)

        def piece_body(q, cursor):
            pltpu.sync_copy(idx_hbm.at[pl.ds(q * 2048, 2048)], idx_v)

            def pre_body(s, cursor):
                vi = idx_v[pl.ds(s * LANES, LANES)]
                m = (vi >= tile_lo) & (vi < tile_hi)
                cnt = _scalar(plsc.all_reduce_population_count(m), 0, iota)
                packed = lax.shift_left(vi - tile_lo, 14) | \
                    (iota + q * 2048 + s * LANES)
                plsc.store_compressed(lpk_v.at[pl.ds(cursor, LANES)], packed,
                                      mask=m)
                return cursor + cnt

            return lax.fori_loop(0, 2048 // LANES, pre_body, cursor)

        nlist = lax.fori_loop(0, BATCH // 2048, piece_body, 0)
        nsteps = (nlist + LANES - 1) // LANES

        def process_chunk(clo, chi, width, parity, fired):
            rlo = clo - tile_lo
            rhi = chi - tile_lo

            def step_body(s, fired):
                pk = lpk_v[pl.ds(s * LANES, LANES)]
                m = (pk >= rlo * (1 << 14)) & (pk < rhi * (1 << 14))
                m = m & (s * LANES + iota < nlist)
                cnt = _scalar(plsc.all_reduce_population_count(m), 0, iota)
                plsc.store_compressed(tmp_v.at[...], pk, mask=m)

                def entry_body(e2, fired):
                    pks = _scalar(tmp_v[...], e2, iota)
                    dr = lax.shift_right_logical(pks, 14) - rlo
                    pos = pks & (BATCH - 1)
                    parityv = jnp.full((LANES,), parity, jnp.int32)
                    drv = jnp.full((LANES,), dr, jnp.int32)
                    g0 = plsc.load_gather(cbuf, [parityv, iota, drv])
                    g1 = plsc.load_gather(cbuf, [parityv, iota + LANES, drv])
                    gb = plsc.load_gather(
                        bbuf, [parityv, jnp.zeros((LANES,), jnp.int32), drv])
                    return _scatter_entry(g0, g1, gb, pos, fired)

                return lax.fori_loop(0, cnt, entry_body, fired)

            return lax.fori_loop(0, nsteps, step_body, fired)

        def chunk_pair(p, fired):
            for parity in (0, 1):
                c = 2 * p + parity
                lo = tile_lo + c * CW
                drain(lo, CW, parity)
                fired = process_chunk(lo, lo + CW, CW, parity, fired)

                @pl.when(c + 2 < NFULL - 1)
                def _():
                    fire(lo + 2 * CW, CW, parity)
            return fired

        fired = lax.fori_loop(0, (NFULL - 1) // 2, chunk_pair, 0)
        lo68 = tile_lo + (NFULL - 1) * CW
        fire(lo68, CW, 0)
        drain(lo68, CW, 0)
        fired = process_chunk(lo68, lo68 + CW, CW, 0, fired)
        lo69 = tile_lo + NFULL * CW
        fire(lo69, LASTW, 1)
        drain(lo69, LASTW, 1)
        fired = process_chunk(lo69, lo69 + LASTW, LASTW, 1, fired)

        def process_tail(fired):
            rlo = TAIL_LO - tile_lo

            def step_body(s, fired):
                pk = lpk_v[pl.ds(s * LANES, LANES)]
                m = (pk >= rlo * (1 << 14)) & (s * LANES + iota < nlist)
                cnt = _scalar(plsc.all_reduce_population_count(m), 0, iota)
                plsc.store_compressed(tmp_v.at[...], pk, mask=m)

                def entry_body(e2, fired):
                    pks = _scalar(tmp_v[...], e2, iota)
                    dr = lax.shift_right_logical(pks, 14) - rlo
                    pos = pks & (BATCH - 1)
                    g0 = tail_v[dr, pl.ds(0, LANES)]
                    g1 = tail_v[dr, pl.ds(LANES, LANES)]
                    gb = plsc.load_gather(
                        tailb_v, [jnp.full((LANES,), dr, jnp.int32)])
                    return _scatter_entry(g0, g1, gb, pos, fired)

                return lax.fori_loop(0, cnt, entry_body, fired)

            return lax.fori_loop(0, nsteps, step_body, fired)

        @pl.when(tid == NUM_SUBCORES - 1)
        def _tail():
            pltpu.sync_copy(ttab_hbm, tail_v)
            pltpu.sync_copy(tbias_hbm, tailb_v)
            f2 = fired
            fire(999424, 512, 1)
            drain(999424, 512, 1)
            f2 = process_chunk(999424, TAIL_LO, 512, 1, f2)
            f2 = process_tail(f2)
            _drain_scatters(f2)

        @pl.when(tid != NUM_SUBCORES - 1)
        def _notail():
            _drain_scatters(fired)

    def _drain_scatters(fired):
        del fired

    @pl.when(cid == 0)
    def _user():
        run_side(user_hbm, uet_hbm, ubt_hbm, ut_hbm, ubt2_hbm)

    @pl.when(cid == 1)
    def _item():
        run_side(item_hbm, iet_hbm, ibt_hbm, it_hbm, ibt2_hbm)

    plsc.subcore_barrier()

    for k in range(2):
        r = tid * 2 + k
        pltpu.sync_copy(sh.at[pl.ds(r * BATCH, BATCH)], out_hbm.at[cid, r],
                        )
    @pl.when(tid == NUM_SUBCORES - 1)
    def _bias_row():
        pltpu.sync_copy(sh.at[pl.ds(EMBED * BATCH, BATCH)],
                        out_hbm.at[cid, EMBED])


B_PER_W = BATCH // (NUM_CORES * NUM_SUBCORES)


@functools.partial(
    pl.kernel,
    mesh=plsc.VectorSubcoreMesh(core_axis_name="c", subcore_axis_name="s"),
    out_type=jax.ShapeDtypeStruct((BATCH,), jnp.float32),
    scratch_types=[
        pltpu.VMEM((OROW, B_PER_W), jnp.float32),
        pltpu.VMEM((OROW, B_PER_W), jnp.float32),
        pltpu.VMEM((B_PER_W,), jnp.float32),
        pltpu.SemaphoreType.DMA,
    ],
)
def _mf_dot(gat_hbm, out_hbm, ue_v, ie_v, res_v, sem):
    wid = lax.axis_index("s") * NUM_CORES + lax.axis_index("c")
    base = wid * B_PER_W
    cu = pltpu.async_copy(
        gat_hbm.at[0, pl.ds(0, OROW), pl.ds(base, B_PER_W)], ue_v, sem)
    ci = pltpu.async_copy(
        gat_hbm.at[1, pl.ds(0, OROW), pl.ds(base, B_PER_W)], ie_v, sem)
    cu.wait()
    ci.wait()

    def group_body(g, carry):
        sl = pl.ds(g * LANES, LANES)
        acc = ue_v[EMBED, sl] + ie_v[EMBED, sl]
        for e in range(EMBED):
            acc = acc + ue_v[e, sl] * ie_v[e, sl]
        res_v[sl] = 1.0 / (1.0 + jnp.exp(-acc))
        return carry

    lax.fori_loop(0, B_PER_W // LANES, group_body, 0)
    pltpu.sync_copy(res_v, out_hbm.at[pl.ds(base, B_PER_W)])


def kernel(user, item, user_e, item_e, user_b, item_b):
    gathered = _mf_gather(user, item, user_e.T, item_e.T,
                          user_b.T, item_b.T,
                          user_e[TAIL_LO:], item_e[TAIL_LO:],
                          user_b[TAIL_LO:].reshape(-1),
                          item_b[TAIL_LO:].reshape(-1))
    return _mf_dot(gathered)
